# baseline (speedup 1.0000x reference)
import functools

import jax
import jax.numpy as jnp
from jax import lax
from jax.experimental import pallas as pl
from jax.experimental.pallas import tpu as pltpu
from jax.experimental.pallas import tpu_sc as plsc

H = 800.0
W = 800.0
SAMPLING = 16.0
TOPN_NMS = 12000
N_NMS = 2000
THR_NMS = 0.7
N = 20000


def _decode_body(anchor_ref, delta_ref, score_ref, roi_ref, score_m_ref):
    a0 = anchor_ref[0, :]
    a1 = anchor_ref[1, :]
    a2 = anchor_ref[2, :]
    a3 = anchor_ref[3, :]
    d0 = delta_ref[0, :]
    d1 = delta_ref[1, :]
    d2 = delta_ref[2, :]
    d3 = delta_ref[3, :]

    anc_w = a2 - a0
    anc_h = a3 - a1
    anc_ctrx = a0 + anc_w / 2.0
    anc_ctry = a1 + anc_h / 2.0
    ctr_x = d0 * anc_w + anc_ctrx
    ctr_y = d1 * anc_h + anc_ctry
    w = jnp.exp(d2) * anc_w
    h = jnp.exp(d3) * anc_h

    x1 = jnp.maximum(ctr_x - w / 2.0, 0.0)
    y1 = jnp.maximum(ctr_y - h / 2.0, 0.0)
    x2 = jnp.maximum(ctr_x + w / 2.0, 0.0)
    y2 = jnp.maximum(ctr_y + h / 2.0, 0.0)
    y2 = jnp.where(y2 > H, H - 1.0, y2)
    x2 = jnp.where(x2 > W, W - 1.0, x2)

    valid = ((x2 - x1) > SAMPLING) & ((y2 - y1) > SAMPLING)
    roi_ref[0, :] = x1
    roi_ref[1, :] = y1
    roi_ref[2, :] = x2
    roi_ref[3, :] = y2
    score_m_ref[0, :] = jnp.where(valid, score_ref[0, :], -jnp.inf)


@jax.jit
def _decode(anchor_t, delta_t, score_2d):
    n = anchor_t.shape[1]
    return pl.pallas_call(
        _decode_body,
        out_shape=(
            jax.ShapeDtypeStruct((4, n), jnp.float32),
            jax.ShapeDtypeStruct((1, n), jnp.float32),
        ),
    )(anchor_t, delta_t, score_2d)


def _make_nms(npad, tile, n_sel=2048, interpret=False):
    nt = npad // tile

    def body(boxes_t_ref, boxes_c_ref, sel_ref, act_r, act_c, cond_ref,
             tri_ref, cinc_ref):
        act_r[...] = boxes_t_ref[...]
        act_c[...] = boxes_c_ref[...]
        tri_ref[...] = jnp.where(
            lax.broadcasted_iota(jnp.int32, (tile, tile), 0)
            <= lax.broadcasted_iota(jnp.int32, (tile, tile), 1), 1.0, 0.0)

        def tile_body(i, carry):
            sl = pl.ds(i * tile, tile)
            xi1 = act_r[0:1, sl]
            yi1 = act_r[1:2, sl]
            xi2 = act_r[2:3, sl]
            yi2 = act_r[3:4, sl]
            ai = (xi2 - xi1) * (yi2 - yi1)
            ti1 = act_c[sl, 0:1]
            ti2 = act_c[sl, 1:2]
            ti3 = act_c[sl, 2:3]
            ti4 = act_c[sl, 3:4]
            ta = (ti3 - ti1) * (ti4 - ti2)

            def cross(j, sup):
                sj = pl.ds(j * tile, tile)
                xj1 = act_c[sj, 0:1]
                yj1 = act_c[sj, 1:2]
                xj2 = act_c[sj, 2:3]
                yj2 = act_c[sj, 3:4]
                aj = (xj2 - xj1) * (yj2 - yj1)
                xx1 = jnp.maximum(xj1, xi1)
                yy1 = jnp.maximum(yj1, yi1)
                xx2 = jnp.minimum(xj2, xi2)
                yy2 = jnp.minimum(yj2, yi2)
                inter = (jnp.maximum(xx2 - xx1, 0.0)
                         * jnp.maximum(yy2 - yy1, 0.0))
                iou = inter / (aj + ai - inter + 1e-9)
                hit = jnp.where(iou >= THR_NMS, 1.0, 0.0)
                return jnp.maximum(sup, jnp.max(hit, axis=0)[None, :])

            sup0 = lax.fori_loop(0, i, cross,
                                 jnp.zeros((1, tile), jnp.float32))

            xx1 = jnp.maximum(ti1, xi1)
            yy1 = jnp.maximum(ti2, yi1)
            xx2 = jnp.minimum(ti3, xi2)
            yy2 = jnp.minimum(ti4, yi2)
            inter = jnp.maximum(xx2 - xx1, 0.0) * jnp.maximum(yy2 - yy1, 0.0)
            iou = inter / (ta + ai - inter + 1e-9)
            rlt = (lax.broadcasted_iota(jnp.int32, (tile, tile), 0)
                   < lax.broadcasted_iota(jnp.int32, (tile, tile), 1))
            cond_ref[...] = jnp.where((iou >= THR_NMS) & rlt, 1.0, 0.0)

            def witer(wcarry):
                s, _ = wcarry
                active_col = (1.0 - s)[0][:, None]
                m = jnp.max(cond_ref[...] * active_col, axis=0)[None, :]
                s_new = jnp.maximum(sup0, m)
                changed = jnp.max(jnp.abs(s_new - s)) > 0.0
                return (s_new, changed)

            s_final, _ = lax.while_loop(lambda c: c[1], witer,
                                        (sup0, jnp.bool_(True)))

            gmask = (lax.broadcasted_iota(jnp.int32, (1, tile), 1)
                     + i * tile) < TOPN_NMS
            keepf = jnp.where(gmask & (s_final < 0.5), 1.0, 0.0)
            incl = lax.dot_general(keepf, tri_ref[...],
                                   (((1,), (0,)), ((), ())),
                                   preferred_element_type=jnp.float32)
            cinc_ref[0:1, sl] = carry + incl
            new_carry = carry + jnp.sum(keepf)
            fac_r = 1.0 - s_final
            fac_c = fac_r[0][:, None]
            act_r[0:1, sl] = xi1 * fac_r
            act_r[1:2, sl] = yi1 * fac_r
            act_r[2:3, sl] = xi2 * fac_r
            act_r[3:4, sl] = yi2 * fac_r
            act_c[sl, 0:1] = ti1 * fac_c
            act_c[sl, 1:2] = ti2 * fac_c
            act_c[sl, 2:3] = ti3 * fac_c
            act_c[sl, 3:4] = ti4 * fac_c
            return new_carry

        total = lax.fori_loop(0, nt, tile_body, jnp.float32(0.0))

        jio = lax.broadcasted_iota(jnp.int32, (n_sel, 1), 0).astype(jnp.float32)

        def sel_body(t, acc):
            cr = cinc_ref[0:1, pl.ds(t * tile, tile)]
            hit = jnp.where(cr <= jio, 1.0, 0.0)
            return acc + jnp.sum(hit, axis=1, keepdims=True)

        acc = lax.fori_loop(0, nt, sel_body,
                            jnp.zeros((n_sel, 1), jnp.float32))
        sel_ref[...] = jnp.where(jio < total, acc, 0.0).astype(jnp.int32)

    def call(boxes_t, boxes_c):
        return pl.pallas_call(
            body,
            out_shape=jax.ShapeDtypeStruct((n_sel, 1), jnp.int32),
            scratch_shapes=[
                pltpu.VMEM((4, npad), jnp.float32),
                pltpu.VMEM((npad, 4), jnp.float32),
                pltpu.VMEM((tile, tile), jnp.float32),
                pltpu.VMEM((tile, tile), jnp.float32),
                pltpu.VMEM((1, npad), jnp.float32),
            ],
            interpret=interpret,
        )(boxes_t, boxes_c)

    return call


def _nms_keep_ref(boxes, thr):
    n = boxes.shape[0]
    areas = (boxes[:, 2] - boxes[:, 0]) * (boxes[:, 3] - boxes[:, 1])
    idxs = jnp.arange(n)

    def body(i, suppressed):
        xx1 = jnp.maximum(boxes[i, 0], boxes[:, 0])
        yy1 = jnp.maximum(boxes[i, 1], boxes[:, 1])
        xx2 = jnp.minimum(boxes[i, 2], boxes[:, 2])
        yy2 = jnp.minimum(boxes[i, 3], boxes[:, 3])
        inter = jnp.maximum(xx2 - xx1, 0.0) * jnp.maximum(yy2 - yy1, 0.0)
        iou = inter / (areas[i] + areas - inter + 1e-9)
        new_sup = (iou >= thr) & (idxs > i)
        return jnp.where(suppressed[i], suppressed, suppressed | new_sup)

    suppressed = jax.lax.fori_loop(0, n, body, jnp.zeros((n,), dtype=bool))
    return ~suppressed


NPAD_NMS = 12288
TILE_NMS = 512
_nms_call = _make_nms(NPAD_NMS, TILE_NMS)


N_SEL = 2048


def _make_compact(npad, n_sel, interpret=False):
    mesh = plsc.VectorSubcoreMesh(core_axis_name="c", subcore_axis_name="s")
    chunk = n_sel // 32

    @functools.partial(
        pl.kernel,
        out_type=jax.ShapeDtypeStruct((n_sel, 128), jnp.float32),
        mesh=mesh,
        interpret=interpret,
        scratch_types=[
            pltpu.VMEM((chunk,), jnp.int32),
            pltpu.VMEM((chunk, 128), jnp.float32),
            pltpu.SemaphoreType.DMA,
        ],
    )
    def k(sel_hbm, boxes_hbm, out_hbm, sel_v, row_v, sem):
        wid = lax.axis_index("s") * 2 + lax.axis_index("c")
        base = wid * chunk
        pltpu.sync_copy(sel_hbm.at[pl.ds(base, chunk)], sel_v)
        pltpu.async_copy(boxes_hbm.at[sel_v], row_v, sem).wait()
        pltpu.sync_copy(row_v, out_hbm.at[pl.ds(base, chunk), :])

    return k


def kernel(anchor, delta, score):
    roi_t, score_m = _decode(anchor.T, delta.T, score[None, :])
    score_m = score_m[0]
    order = jnp.argsort(-score_m)[:TOPN_NMS]
    boxes = roi_t.T[order]
    boxes_pad = jnp.zeros((NPAD_NMS, 4), jnp.float32).at[:TOPN_NMS].set(boxes)
    boxes_pad_t = boxes_pad.T
    sel2d = _nms_call(boxes_pad_t, boxes_pad)
    boxes128 = jnp.pad(boxes_pad, ((0, 0), (0, 124)))
    outp = _make_compact(NPAD_NMS, N_SEL)(sel2d.reshape(N_SEL), boxes128)
    return outp[:N_NMS, :4]
